# baseline (speedup 1.0000x reference)
import functools
import math

import jax
import jax.numpy as jnp
from jax import lax
from jax.experimental import pallas as pl
from jax.experimental.pallas import tpu as pltpu
from jax.experimental.pallas import tpu_sc as plsc

N = 4
C = 64
CPAD = 128
S = 512
P = 196
PPAD = 256
K = 12
BIG = 3.0e38
_EXPW = [math.exp(-k) for k in range(K)]

NC = 2
NS = 16
POS_PER_TILE = 32


def _tc_distance_body(x_ref, cb_ref, d_ref, idx_ref, m0_ref, cbp_ref,
                      lcm_ref):
    n = pl.program_id(0)
    cb = cb_ref[...]
    cbp_ref[...] = jnp.concatenate(
        [cb, jnp.zeros((S, CPAD - C), jnp.float32)], axis=1)
    cb2 = jnp.sum(cb * cb, axis=1, keepdims=True)
    iota_s = jax.lax.broadcasted_iota(jnp.int32, (S, P), 0)
    xn = x_ref[0]
    xn2 = jnp.sum(xn * xn, axis=0, keepdims=True)
    dot = jax.lax.dot_general(cb, xn, (((1,), (0,)), ((), ())),
                              precision=jax.lax.Precision.HIGHEST,
                              preferred_element_type=jnp.float32)
    d = cb2 - 2.0 * dot + xn2
    d_ref[0] = d
    m = jnp.min(d, axis=0, keepdims=True)
    fidx = jnp.min(jnp.where(d == m, iota_s, S), axis=0, keepdims=True)
    fpad = jnp.concatenate(
        [fidx, jnp.zeros((1, PPAD - P), jnp.int32)], axis=1)
    idx_ref[pl.ds(2 * n, 2), :] = fpad.reshape(2, CPAD)
    m0_ref[pl.ds(n, 1), :] = m

    @pl.when(n == 0)
    def _():
        lcm_ref[0, 0] = 0.0

    lcm_ref[0, 0] += jnp.sum(m) / jnp.float32(N * C * P)


def _tc_loss_body(d_ref, m0_ref, lcb_ref):
    n = pl.program_id(0)
    d = d_ref[0]
    mk = m0_ref[pl.ds(n, 1), :]
    lcb = jnp.float32(0.0)
    for k in range(K):
        lcb = lcb + jnp.float32(_EXPW[k]) * jnp.sum(mk)
        if k < K - 1:
            mk = jnp.min(jnp.where(d > mk, d, BIG), axis=0, keepdims=True)

    @pl.when(n == 0)
    def _():
        lcb_ref[0, 0] = 0.0

    lcb_ref[0, 0] += lcb / jnp.float32(N * S * P)


def _sc_gather_body(idx_hbm, cbp_hbm, xq_hbm, idxv, rowsv, sem):
    wid = lax.axis_index("s") * NC + lax.axis_index("c")
    pltpu.sync_copy(idx_hbm, idxv)
    row = wid // 4
    off = (wid % 4) * POS_PER_TILE
    i0 = idxv[row, pl.ds(off, 16)]
    i1 = idxv[row, pl.ds(off + 16, 16)]
    cp0 = pltpu.async_copy(cbp_hbm.at[i0], rowsv.at[pl.ds(0, 16), :], sem)
    cp1 = pltpu.async_copy(cbp_hbm.at[i1], rowsv.at[pl.ds(16, 16), :], sem)
    cp0.wait()
    cp1.wait()
    pltpu.sync_copy(rowsv, xq_hbm.at[pl.ds(POS_PER_TILE * wid,
                                           POS_PER_TILE), :])


@functools.partial(
    pl.kernel,
    out_type=jax.ShapeDtypeStruct((N * PPAD, CPAD), jnp.float32),
    mesh=plsc.VectorSubcoreMesh(
        core_axis_name="c", subcore_axis_name="s",
        num_cores=NC, num_subcores=NS),
    compiler_params=pltpu.CompilerParams(needs_layout_passes=False),
    scratch_types=[
        pltpu.VMEM((2 * N, CPAD), jnp.int32),
        pltpu.VMEM((POS_PER_TILE, CPAD), jnp.float32),
        pltpu.SemaphoreType.DMA,
    ],
)
def _sc_gather(idx_hbm, cbp_hbm, xq_hbm, idxv, rowsv, sem):
    _sc_gather_body(idx_hbm, cbp_hbm, xq_hbm, idxv, rowsv, sem)


def kernel(x, codebook):
    x3 = x.reshape(N, C, P)
    d, idx, m0, cbp, lcm = pl.pallas_call(
        _tc_distance_body,
        grid=(N,),
        out_shape=(
            jax.ShapeDtypeStruct((N, S, P), jnp.float32),
            jax.ShapeDtypeStruct((2 * N, CPAD), jnp.int32),
            jax.ShapeDtypeStruct((N, P), jnp.float32),
            jax.ShapeDtypeStruct((S, CPAD), jnp.float32),
            jax.ShapeDtypeStruct((1, 1), jnp.float32),
        ),
        out_specs=(
            pl.BlockSpec((1, S, P), lambda n: (n, 0, 0)),
            pl.BlockSpec((2 * N, CPAD), lambda n: (0, 0)),
            pl.BlockSpec((N, P), lambda n: (0, 0)),
            pl.BlockSpec((S, CPAD), lambda n: (0, 0)),
            pl.BlockSpec(memory_space=pltpu.SMEM),
        ),
        in_specs=(
            pl.BlockSpec((1, C, P), lambda n: (n, 0, 0)),
            pl.BlockSpec((S, C), lambda n: (0, 0)),
        ),
    )(x3, codebook)

    xq = _sc_gather(idx, cbp)

    lcb = pl.pallas_call(
        _tc_loss_body,
        grid=(N,),
        out_shape=jax.ShapeDtypeStruct((1, 1), jnp.float32),
        out_specs=pl.BlockSpec(memory_space=pltpu.SMEM),
        in_specs=(
            pl.BlockSpec((1, S, P), lambda n: (n, 0, 0)),
            pl.BlockSpec((N, P), lambda n: (0, 0)),
        ),
    )(d, m0)

    xq4 = xq.reshape(N, PPAD, CPAD)[:, :P, :C]
    output = xq4.transpose(0, 2, 1).reshape(x.shape)
    idx4 = idx.reshape(N, PPAD)[:, :P]
    return (output, lcb[0, 0], lcm[0, 0], idx4.reshape(N, 14, 14))


# speedup vs baseline: 1.0236x; 1.0236x over previous
import functools
import math

import jax
import jax.numpy as jnp
from jax import lax
from jax.experimental import pallas as pl
from jax.experimental.pallas import tpu as pltpu
from jax.experimental.pallas import tpu_sc as plsc

N = 4
C = 64
CPAD = 128
S = 512
P = 196
PPAD = 256
K = 12
BIG = 3.0e38
_EXPW = [math.exp(-k) for k in range(K)]

NC = 2
NS = 16
POS_PER_TILE = 32


def _tc_distance_body(x_ref, cb_ref, d_ref, idx_ref, m0_ref, cbp_ref,
                      lcm_ref):
    cb = cb_ref[...]
    cbp_ref[...] = jnp.concatenate(
        [cb, jnp.zeros((S, CPAD - C), jnp.float32)], axis=1)
    cb2 = jnp.sum(cb * cb, axis=1, keepdims=True)
    iota_s = jax.lax.broadcasted_iota(jnp.int32, (S, P), 0)
    zpad = jnp.zeros((1, PPAD - P), jnp.int32)
    lcm = jnp.float32(0.0)
    for n in range(N):
        xn = x_ref[n]
        xn2 = jnp.sum(xn * xn, axis=0, keepdims=True)
        dot = jax.lax.dot_general(cb, xn, (((1,), (0,)), ((), ())),
                                  precision=jax.lax.Precision.HIGHEST,
                                  preferred_element_type=jnp.float32)
        d = cb2 - 2.0 * dot + xn2
        d_ref[n] = d
        m = jnp.min(d, axis=0, keepdims=True)
        fidx = jnp.min(jnp.where(d == m, iota_s, S), axis=0, keepdims=True)
        fpad = jnp.concatenate([fidx, zpad], axis=1)
        idx_ref[pl.ds(2 * n, 2), :] = fpad.reshape(2, CPAD)
        m0_ref[pl.ds(n, 1), :] = m
        lcm = lcm + jnp.sum(m)
    lcm_ref[0, 0] = lcm / jnp.float32(N * C * P)


def _tc_loss_body(d_ref, m0_ref, lcb_ref):
    lcb = jnp.float32(0.0)
    for n in range(N):
        d = d_ref[n]
        mk = m0_ref[pl.ds(n, 1), :]
        for k in range(K):
            lcb = lcb + jnp.float32(_EXPW[k]) * jnp.sum(mk)
            if k < K - 1:
                mk = jnp.min(jnp.where(d > mk, d, BIG), axis=0,
                             keepdims=True)
    lcb_ref[0, 0] = lcb / jnp.float32(N * S * P)


def _sc_gather_body(idx_hbm, cbp_hbm, xq_hbm, idxv, idx32, rowsv, sem):
    wid = lax.axis_index("s") * NC + lax.axis_index("c")
    pltpu.sync_copy(idx_hbm, idxv)
    row = wid // 4
    off = (wid % 4) * POS_PER_TILE
    idx32[pl.ds(0, 16)] = idxv[row, pl.ds(off, 16)]
    idx32[pl.ds(16, 16)] = idxv[row, pl.ds(off + 16, 16)]
    pltpu.async_copy(cbp_hbm.at[idx32], rowsv, sem).wait()
    pltpu.sync_copy(rowsv, xq_hbm.at[pl.ds(POS_PER_TILE * wid,
                                           POS_PER_TILE), :])


@functools.partial(
    pl.kernel,
    out_type=jax.ShapeDtypeStruct((N * PPAD, CPAD), jnp.float32),
    mesh=plsc.VectorSubcoreMesh(
        core_axis_name="c", subcore_axis_name="s",
        num_cores=NC, num_subcores=NS),
    compiler_params=pltpu.CompilerParams(
        needs_layout_passes=False, use_tc_tiling_on_sc=True),
    scratch_types=[
        pltpu.VMEM((2 * N, CPAD), jnp.int32),
        pltpu.VMEM((POS_PER_TILE,), jnp.int32),
        pltpu.VMEM((POS_PER_TILE, CPAD), jnp.float32),
        pltpu.SemaphoreType.DMA,
    ],
)
def _sc_gather(idx_hbm, cbp_hbm, xq_hbm, idxv, idx32, rowsv, sem):
    _sc_gather_body(idx_hbm, cbp_hbm, xq_hbm, idxv, idx32, rowsv, sem)


def kernel(x, codebook):
    x3 = x.reshape(N, C, P)
    d, idx, m0, cbp, lcm = pl.pallas_call(
        _tc_distance_body,
        out_shape=(
            jax.ShapeDtypeStruct((N, S, P), jnp.float32),
            jax.ShapeDtypeStruct((2 * N, CPAD), jnp.int32),
            jax.ShapeDtypeStruct((N, P), jnp.float32),
            jax.ShapeDtypeStruct((S, CPAD), jnp.float32),
            jax.ShapeDtypeStruct((1, 1), jnp.float32),
        ),
        out_specs=(
            pl.BlockSpec(memory_space=pltpu.VMEM),
            pl.BlockSpec(memory_space=pltpu.VMEM),
            pl.BlockSpec(memory_space=pltpu.VMEM),
            pl.BlockSpec(memory_space=pltpu.VMEM),
            pl.BlockSpec(memory_space=pltpu.SMEM),
        ),
        in_specs=(
            pl.BlockSpec(memory_space=pltpu.VMEM),
            pl.BlockSpec(memory_space=pltpu.VMEM),
        ),
    )(x3, codebook)

    xq = _sc_gather(idx, cbp)

    lcb = pl.pallas_call(
        _tc_loss_body,
        out_shape=jax.ShapeDtypeStruct((1, 1), jnp.float32),
        out_specs=pl.BlockSpec(memory_space=pltpu.SMEM),
        in_specs=(
            pl.BlockSpec(memory_space=pltpu.VMEM),
            pl.BlockSpec(memory_space=pltpu.VMEM),
        ),
    )(d, m0)

    xq4 = xq.reshape(N, PPAD, CPAD)[:, :P, :C]
    output = xq4.transpose(0, 2, 1).reshape(x.shape)
    idx4 = idx.reshape(N, PPAD)[:, :P]
    return (output, lcb[0, 0], lcm[0, 0], idx4.reshape(N, 14, 14))


# speedup vs baseline: 1.0609x; 1.0365x over previous
import functools
import math

import jax
import jax.numpy as jnp
from jax import lax
from jax.experimental import pallas as pl
from jax.experimental.pallas import tpu as pltpu
from jax.experimental.pallas import tpu_sc as plsc

N = 4
C = 64
CPAD = 128
S = 512
P = 196
PPAD = 256
K = 12
BIG = 3.0e38
_EXPW = [math.exp(-k) for k in range(K)]

NC = 2
NS = 16
POS_PER_TILE = 32


def _distances(cb, xn):
    cb2 = jnp.sum(cb * cb, axis=1, keepdims=True)
    xn2 = jnp.sum(xn * xn, axis=0, keepdims=True)
    dot = jax.lax.dot_general(cb, xn, (((1,), (0,)), ((), ())),
                              precision=jax.lax.Precision.HIGHEST,
                              preferred_element_type=jnp.float32)
    return cb2 - 2.0 * dot + xn2


def _tc_distance_body(x_ref, cb_ref, idx_ref, m0_ref, cbp_ref, lcm_ref):
    cb = cb_ref[...]
    cbp_ref[...] = jnp.concatenate(
        [cb, jnp.zeros((S, CPAD - C), jnp.float32)], axis=1)
    iota_s = jax.lax.broadcasted_iota(jnp.int32, (S, P), 0)
    zpad = jnp.zeros((1, PPAD - P), jnp.int32)
    lcm = jnp.float32(0.0)
    for n in range(N):
        d = _distances(cb, x_ref[n])
        m = jnp.min(d, axis=0, keepdims=True)
        fidx = jnp.min(jnp.where(d == m, iota_s, S), axis=0, keepdims=True)
        fpad = jnp.concatenate([fidx, zpad], axis=1)
        idx_ref[pl.ds(2 * n, 2), :] = fpad.reshape(2, CPAD)
        m0_ref[pl.ds(n, 1), :] = m
        lcm = lcm + jnp.sum(m)
    lcm_ref[0, 0] = lcm / jnp.float32(N * C * P)


def _tc_loss_body(x_ref, cb_ref, m0_ref, lcb_ref):
    cb = cb_ref[...]
    lcb = jnp.float32(0.0)
    for n in range(N):
        d = _distances(cb, x_ref[n])
        mk = m0_ref[pl.ds(n, 1), :]
        for k in range(K):
            lcb = lcb + jnp.float32(_EXPW[k]) * jnp.sum(mk)
            if k < K - 1:
                mk = jnp.min(jnp.where(d > mk, d, BIG), axis=0,
                             keepdims=True)
    lcb_ref[0, 0] = lcb / jnp.float32(N * S * P)


def _sc_gather_body(idx_hbm, cbp_hbm, xq_hbm, idxv, idx32, rowsv, sem):
    wid = lax.axis_index("s") * NC + lax.axis_index("c")
    pltpu.sync_copy(idx_hbm, idxv)
    row = wid // 4
    off = (wid % 4) * POS_PER_TILE
    idx32[pl.ds(0, 16)] = idxv[row, pl.ds(off, 16)]
    idx32[pl.ds(16, 16)] = idxv[row, pl.ds(off + 16, 16)]
    pltpu.async_copy(cbp_hbm.at[idx32], rowsv, sem).wait()
    pltpu.sync_copy(rowsv, xq_hbm.at[pl.ds(POS_PER_TILE * wid,
                                           POS_PER_TILE), :])


@functools.partial(
    pl.kernel,
    out_type=jax.ShapeDtypeStruct((N * PPAD, CPAD), jnp.float32),
    mesh=plsc.VectorSubcoreMesh(
        core_axis_name="c", subcore_axis_name="s",
        num_cores=NC, num_subcores=NS),
    compiler_params=pltpu.CompilerParams(
        needs_layout_passes=False, use_tc_tiling_on_sc=True),
    scratch_types=[
        pltpu.VMEM((2 * N, CPAD), jnp.int32),
        pltpu.VMEM((POS_PER_TILE,), jnp.int32),
        pltpu.VMEM((POS_PER_TILE, CPAD), jnp.float32),
        pltpu.SemaphoreType.DMA,
    ],
)
def _sc_gather(idx_hbm, cbp_hbm, xq_hbm, idxv, idx32, rowsv, sem):
    _sc_gather_body(idx_hbm, cbp_hbm, xq_hbm, idxv, idx32, rowsv, sem)


def kernel(x, codebook):
    x3 = x.reshape(N, C, P)
    idx, m0, cbp, lcm = pl.pallas_call(
        _tc_distance_body,
        out_shape=(
            jax.ShapeDtypeStruct((2 * N, CPAD), jnp.int32),
            jax.ShapeDtypeStruct((N, P), jnp.float32),
            jax.ShapeDtypeStruct((S, CPAD), jnp.float32),
            jax.ShapeDtypeStruct((1, 1), jnp.float32),
        ),
        out_specs=(
            pl.BlockSpec(memory_space=pltpu.VMEM),
            pl.BlockSpec(memory_space=pltpu.VMEM),
            pl.BlockSpec(memory_space=pltpu.VMEM),
            pl.BlockSpec(memory_space=pltpu.SMEM),
        ),
        in_specs=(
            pl.BlockSpec(memory_space=pltpu.VMEM),
            pl.BlockSpec(memory_space=pltpu.VMEM),
        ),
    )(x3, codebook)

    xq = _sc_gather(idx, cbp)

    lcb = pl.pallas_call(
        _tc_loss_body,
        out_shape=jax.ShapeDtypeStruct((1, 1), jnp.float32),
        out_specs=pl.BlockSpec(memory_space=pltpu.SMEM),
        in_specs=(
            pl.BlockSpec(memory_space=pltpu.VMEM),
            pl.BlockSpec(memory_space=pltpu.VMEM),
            pl.BlockSpec(memory_space=pltpu.VMEM),
        ),
    )(x3, codebook, m0)

    xq4 = xq.reshape(N, PPAD, CPAD)[:, :P, :C]
    output = xq4.transpose(0, 2, 1).reshape(x.shape)
    idx4 = idx.reshape(N, PPAD)[:, :P]
    return (output, lcb[0, 0], lcm[0, 0], idx4.reshape(N, 14, 14))
